# Initial kernel scaffold; baseline (speedup 1.0000x reference)
import jax
import jax.numpy as jnp
from jax.experimental import pallas as pl


def kernel(input_ids, emb_table, pooler_w, pooler_b):
    raise NotImplementedError("write your pallas kernel here")


# speedup vs baseline: 3.9061x; 3.9061x over previous
import functools

import jax
import jax.numpy as jnp
from jax import lax
from jax.experimental import pallas as pl
from jax.experimental.pallas import tpu as pltpu
from jax.experimental.pallas import tpu_sc as plsc

VOCAB = 100000
HIDDEN = 128
BATCH = 1024
SEQ = 200

NC = 2
NS = 16
NW = NC * NS

TOTAL = BATCH * SEQ
CHUNK = 128
PER_W = TOTAL // NW
N_CHUNKS = PER_W // CHUNK


def _sc_gather(idx3, table):
  mesh = plsc.VectorSubcoreMesh(
      core_axis_name="c", subcore_axis_name="s", num_cores=NC, num_subcores=NS
  )

  @functools.partial(
      pl.kernel,
      out_type=jax.ShapeDtypeStruct((NW, N_CHUNKS, CHUNK, HIDDEN), jnp.float32),
      mesh=mesh,
      scratch_types=[
          pltpu.VMEM((N_CHUNKS, CHUNK), jnp.int32),
          pltpu.VMEM((2, CHUNK, HIDDEN), jnp.float32),
          pltpu.SemaphoreType.DMA,
          pltpu.SemaphoreType.DMA,
          pltpu.SemaphoreType.DMA,
          pltpu.SemaphoreType.DMA,
      ],
  )
  def gather_kernel(idx_hbm, table_hbm, out_hbm, idx_v, rows_v, g0, g1, s0, s1):
    wid = lax.axis_index("s") * NC + lax.axis_index("c")
    pltpu.sync_copy(idx_hbm.at[wid], idx_v)

    gsems = (g0, g1)
    ssems = (s0, s1)

    for b in range(2):
      pltpu.async_copy(table_hbm.at[idx_v.at[b]], rows_v.at[b], gsems[b])

    def group(t, _):
      for b in range(2):
        j = 2 * t + b
        pltpu.make_async_copy(table_hbm.at[idx_v.at[b]], rows_v.at[b],
                              gsems[b]).wait()
        pltpu.async_copy(rows_v.at[b], out_hbm.at[wid, j], ssems[b])

      for b in range(2):
        j = 2 * t + b + 2

        @pl.when(j < N_CHUNKS)
        def _():
          pltpu.make_async_copy(rows_v.at[b], out_hbm.at[wid, j - 2],
                                ssems[b]).wait()
          pltpu.async_copy(table_hbm.at[idx_v.at[j]], rows_v.at[b], gsems[b])

      return 0

    lax.fori_loop(0, N_CHUNKS // 2, group, 0)

    for b in range(2):
      pltpu.make_async_copy(rows_v.at[b], out_hbm.at[wid, N_CHUNKS - 2 + b],
                            ssems[b]).wait()

  return gather_kernel(idx3, table)


def _tc_pooler(seq2, w, b2):
  def pooler_kernel(seq_ref, w_ref, b_ref, out_ref):
    x = seq_ref[...]
    out_ref[...] = (
        lax.dot_general(x, w_ref[...], (((1,), (1,)), ((), ())),
                        preferred_element_type=jnp.float32)
        + b_ref[...]
    )

  return pl.pallas_call(
      pooler_kernel,
      grid=(1,),
      out_shape=jax.ShapeDtypeStruct((BATCH, HIDDEN), jnp.float32),
      in_specs=[
          pl.BlockSpec((BATCH, HIDDEN), lambda i: (0, 0)),
          pl.BlockSpec((HIDDEN, HIDDEN), lambda i: (0, 0)),
          pl.BlockSpec((1, HIDDEN), lambda i: (0, 0)),
      ],
      out_specs=pl.BlockSpec((BATCH, HIDDEN), lambda i: (0, 0)),
  )(seq2, w, b2)


def kernel(input_ids, emb_table, pooler_w, pooler_b):
  idx3 = input_ids.reshape(NW, N_CHUNKS, CHUNK).astype(jnp.int32)
  rows = _sc_gather(idx3, emb_table)
  seq3 = rows.reshape(BATCH, SEQ, HIDDEN)
  pooled = _tc_pooler(rows.reshape(BATCH, SEQ * HIDDEN), pooler_w,
                      pooler_b.reshape(1, HIDDEN))
  return (seq3, pooled)


# speedup vs baseline: 4.0352x; 1.0330x over previous
import functools

import jax
import jax.numpy as jnp
from jax import lax
from jax.experimental import pallas as pl
from jax.experimental.pallas import tpu as pltpu
from jax.experimental.pallas import tpu_sc as plsc

VOCAB = 100000
HIDDEN = 128
BATCH = 1024
SEQ = 200

NC = 2
NS = 16
NW = NC * NS

TOTAL = BATCH * SEQ
CHUNK = 128
PER_W = TOTAL // NW
N_CHUNKS = PER_W // CHUNK


def _sc_gather(idx3, table):
  mesh = plsc.VectorSubcoreMesh(
      core_axis_name="c", subcore_axis_name="s", num_cores=NC, num_subcores=NS
  )
  nbuf = 5
  assert N_CHUNKS % nbuf == 0

  @functools.partial(
      pl.kernel,
      out_type=jax.ShapeDtypeStruct((NW, N_CHUNKS, CHUNK, HIDDEN), jnp.float32),
      mesh=mesh,
      scratch_types=[
          pltpu.VMEM((N_CHUNKS, CHUNK), jnp.int32),
          pltpu.VMEM((nbuf, CHUNK, HIDDEN), jnp.float32),
          [pltpu.SemaphoreType.DMA] * nbuf,
          [pltpu.SemaphoreType.DMA] * nbuf,
      ],
  )
  def gather_kernel(idx_hbm, table_hbm, out_hbm, idx_v, rows_v, gsems, ssems):
    wid = lax.axis_index("s") * NC + lax.axis_index("c")
    pltpu.sync_copy(idx_hbm.at[wid], idx_v)

    for b in range(nbuf):
      pltpu.async_copy(table_hbm.at[idx_v.at[b]], rows_v.at[b], gsems[b])

    def group(t, _):
      for b in range(nbuf):
        j = nbuf * t + b
        pltpu.make_async_copy(table_hbm.at[idx_v.at[b]], rows_v.at[b],
                              gsems[b]).wait()
        pltpu.async_copy(rows_v.at[b], out_hbm.at[wid, j], ssems[b])

      for b in range(nbuf):
        j = nbuf * t + b + nbuf

        @pl.when(j < N_CHUNKS)
        def _():
          pltpu.make_async_copy(rows_v.at[b], out_hbm.at[wid, j - nbuf],
                                ssems[b]).wait()
          pltpu.async_copy(table_hbm.at[idx_v.at[j]], rows_v.at[b], gsems[b])

      return 0

    lax.fori_loop(0, N_CHUNKS // nbuf, group, 0)

    for b in range(nbuf):
      pltpu.make_async_copy(rows_v.at[b], out_hbm.at[wid, N_CHUNKS - nbuf + b],
                            ssems[b]).wait()

  return gather_kernel(idx3, table)


def _tc_pooler(seq2, w, b2):
  def pooler_kernel(seq_ref, w_ref, b_ref, out_ref):
    x = seq_ref[...]
    out_ref[...] = (
        lax.dot_general(x, w_ref[...], (((1,), (1,)), ((), ())),
                        preferred_element_type=jnp.float32)
        + b_ref[...]
    )

  return pl.pallas_call(
      pooler_kernel,
      grid=(1,),
      out_shape=jax.ShapeDtypeStruct((BATCH, HIDDEN), jnp.float32),
      in_specs=[
          pl.BlockSpec((BATCH, HIDDEN), lambda i: (0, 0)),
          pl.BlockSpec((HIDDEN, HIDDEN), lambda i: (0, 0)),
          pl.BlockSpec((1, HIDDEN), lambda i: (0, 0)),
      ],
      out_specs=pl.BlockSpec((BATCH, HIDDEN), lambda i: (0, 0)),
  )(seq2, w, b2)


def kernel(input_ids, emb_table, pooler_w, pooler_b):
  idx3 = input_ids.reshape(NW, N_CHUNKS, CHUNK).astype(jnp.int32)
  rows = _sc_gather(idx3, emb_table)
  seq3 = rows.reshape(BATCH, SEQ, HIDDEN)
  pooled = _tc_pooler(rows.reshape(BATCH, SEQ * HIDDEN), pooler_w,
                      pooler_b.reshape(1, HIDDEN))
  return (seq3, pooled)


# speedup vs baseline: 7.4370x; 1.8430x over previous
import functools

import jax
import jax.numpy as jnp
from jax import lax
from jax.experimental import pallas as pl
from jax.experimental.pallas import tpu as pltpu
from jax.experimental.pallas import tpu_sc as plsc

VOCAB = 100000
HIDDEN = 128
BATCH = 1024
SEQ = 200

NC = 2
NS = 16
NW = NC * NS

CHUNK = 100
B_PER_W = BATCH // NW
N_CHUNKS = B_PER_W * SEQ // CHUNK
HALVES = SEQ // CHUNK


def _sc_gather(idx3, table):
  mesh = plsc.VectorSubcoreMesh(
      core_axis_name="c", subcore_axis_name="s", num_cores=NC, num_subcores=NS
  )
  nbuf = 4
  assert B_PER_W % nbuf == 0

  @functools.partial(
      pl.kernel,
      out_type=jax.ShapeDtypeStruct((BATCH, SEQ, HIDDEN), jnp.float32),
      mesh=mesh,
      scratch_types=[
          pltpu.VMEM((N_CHUNKS, CHUNK), jnp.int32),
          pltpu.VMEM((nbuf, SEQ, HIDDEN), jnp.float32),
          [pltpu.SemaphoreType.DMA] * nbuf,
          [pltpu.SemaphoreType.DMA] * nbuf,
      ],
  )
  def gather_kernel(idx_hbm, table_hbm, out_hbm, idx_v, rows_v, gsems, ssems):
    wid = lax.axis_index("s") * NC + lax.axis_index("c")
    pltpu.sync_copy(idx_hbm.at[wid], idx_v)

    def start_gathers(i, b):
      for h in range(HALVES):
        pltpu.async_copy(table_hbm.at[idx_v.at[HALVES * i + h]],
                         rows_v.at[b, pl.ds(h * CHUNK, CHUNK)], gsems[b])

    def wait_gathers(i, b):
      for h in range(HALVES):
        pltpu.make_async_copy(table_hbm.at[idx_v.at[HALVES * i + h]],
                              rows_v.at[b, pl.ds(h * CHUNK, CHUNK)],
                              gsems[b]).wait()

    def store_sem_op(i, b):
      return pltpu.make_async_copy(rows_v.at[b], out_hbm.at[wid * B_PER_W + i],
                                   ssems[b])

    for b in range(nbuf):
      start_gathers(b, b)

    def group(t, _):
      for b in range(nbuf):
        i = nbuf * t + b
        wait_gathers(i, b)
        store_sem_op(i, b).start()

      for b in range(nbuf):
        i = nbuf * t + b + nbuf

        @pl.when(i < B_PER_W)
        def _():
          store_sem_op(i - nbuf, b).wait()
          start_gathers(i, b)

      return 0

    lax.fori_loop(0, B_PER_W // nbuf, group, 0)

    for b in range(nbuf):
      store_sem_op(B_PER_W - nbuf + b, b).wait()

  return gather_kernel(idx3, table)


def _tc_pooler(seq3, w, b2):
  def pooler_kernel(seq_ref, w_ref, b_ref, out_ref):
    x = seq_ref[:, 0, :]
    out_ref[...] = (
        lax.dot_general(x, w_ref[...], (((1,), (1,)), ((), ())),
                        preferred_element_type=jnp.float32)
        + b_ref[...]
    )

  return pl.pallas_call(
      pooler_kernel,
      grid=(1,),
      out_shape=jax.ShapeDtypeStruct((BATCH, HIDDEN), jnp.float32),
      in_specs=[
          pl.BlockSpec((BATCH, 8, HIDDEN), lambda i: (0, 0, 0)),
          pl.BlockSpec((HIDDEN, HIDDEN), lambda i: (0, 0)),
          pl.BlockSpec((1, HIDDEN), lambda i: (0, 0)),
      ],
      out_specs=pl.BlockSpec((BATCH, HIDDEN), lambda i: (0, 0)),
  )(seq3, w, b2)


def kernel(input_ids, emb_table, pooler_w, pooler_b):
  idx3 = input_ids.reshape(NW, N_CHUNKS, CHUNK).astype(jnp.int32)
  seq3 = _sc_gather(idx3, emb_table)
  pooled = _tc_pooler(seq3, pooler_w, pooler_b.reshape(1, HIDDEN))
  return (seq3, pooled)


# speedup vs baseline: 7.4858x; 1.0066x over previous
import functools

import jax
import jax.numpy as jnp
from jax import lax
from jax.experimental import pallas as pl
from jax.experimental.pallas import tpu as pltpu
from jax.experimental.pallas import tpu_sc as plsc

VOCAB = 100000
HIDDEN = 128
BATCH = 1024
SEQ = 200

NC = 2
NS = 16
NW = NC * NS

CHUNK = 100
B_PER_W = BATCH // NW
N_CHUNKS = B_PER_W * SEQ // CHUNK
HALVES = SEQ // CHUNK


def _sc_gather(idx3, table):
  mesh = plsc.VectorSubcoreMesh(
      core_axis_name="c", subcore_axis_name="s", num_cores=NC, num_subcores=NS
  )
  nbuf = 4
  assert B_PER_W % nbuf == 0

  @functools.partial(
      pl.kernel,
      out_type=jax.ShapeDtypeStruct((BATCH, SEQ, HIDDEN), jnp.float32),
      mesh=mesh,
      scratch_types=[
          pltpu.VMEM((N_CHUNKS, CHUNK), jnp.int32),
          pltpu.VMEM((nbuf, SEQ, HIDDEN), jnp.float32),
          [pltpu.SemaphoreType.DMA] * nbuf,
          [pltpu.SemaphoreType.DMA] * nbuf,
      ],
  )
  def gather_kernel(idx_hbm, table_hbm, out_hbm, idx_v, rows_v, gsems, ssems):
    wid = lax.axis_index("s") * NC + lax.axis_index("c")
    pltpu.sync_copy(idx_hbm.at[wid], idx_v)

    def start_gathers(i, b):
      for h in range(HALVES):
        pltpu.async_copy(table_hbm.at[idx_v.at[HALVES * i + h]],
                         rows_v.at[b, pl.ds(h * CHUNK, CHUNK)], gsems[b])

    def wait_gathers(i, b):
      for h in range(HALVES):
        pltpu.make_async_copy(table_hbm.at[idx_v.at[HALVES * i + h]],
                              rows_v.at[b, pl.ds(h * CHUNK, CHUNK)],
                              gsems[b]).wait()

    def store_sem_op(i, b):
      row = wid * B_PER_W + i
      return (
          pltpu.make_async_copy(rows_v.at[b, pl.ds(0, 104)],
                                out_hbm.at[row, pl.ds(0, 104)], ssems[b]),
          pltpu.make_async_copy(rows_v.at[b, pl.ds(104, 96)],
                                out_hbm.at[row, pl.ds(104, 96)], ssems[b]),
      )

    for b in range(nbuf):
      start_gathers(b, b)

    def group(t, _):
      for b in range(nbuf):
        i = nbuf * t + b
        wait_gathers(i, b)
        for op in store_sem_op(i, b):
          op.start()

      for b in range(nbuf):
        i = nbuf * t + b + nbuf

        @pl.when(i < B_PER_W)
        def _():
          for op in store_sem_op(i - nbuf, b):
            op.wait()
          start_gathers(i, b)

      return 0

    lax.fori_loop(0, B_PER_W // nbuf, group, 0)

    for b in range(nbuf):
      for op in store_sem_op(B_PER_W - nbuf + b, b):
        op.wait()

  return gather_kernel(idx3, table)


def _tc_pooler(seq3, w, b2):
  def pooler_kernel(seq_ref, w_ref, b_ref, out_ref):
    x = seq_ref[:, 0, :]
    out_ref[...] = (
        lax.dot_general(x, w_ref[...], (((1,), (1,)), ((), ())),
                        preferred_element_type=jnp.float32)
        + b_ref[...]
    )

  return pl.pallas_call(
      pooler_kernel,
      grid=(1,),
      out_shape=jax.ShapeDtypeStruct((BATCH, HIDDEN), jnp.float32),
      in_specs=[
          pl.BlockSpec((BATCH, 8, HIDDEN), lambda i: (0, 0, 0)),
          pl.BlockSpec((HIDDEN, HIDDEN), lambda i: (0, 0)),
          pl.BlockSpec((1, HIDDEN), lambda i: (0, 0)),
      ],
      out_specs=pl.BlockSpec((BATCH, HIDDEN), lambda i: (0, 0)),
  )(seq3, w, b2)


def kernel(input_ids, emb_table, pooler_w, pooler_b):
  idx3 = input_ids.reshape(NW, N_CHUNKS, CHUNK).astype(jnp.int32)
  seq3 = _sc_gather(idx3, emb_table)
  pooled = _tc_pooler(seq3, pooler_w, pooler_b.reshape(1, HIDDEN))
  return (seq3, pooled)
